# baseline (speedup 1.0000x reference)
import functools
import math

import jax
import jax.numpy as jnp
from jax import lax
from jax.experimental import pallas as pl
from jax.experimental.pallas import tpu as pltpu
from jax.experimental.pallas import tpu_sc as plsc

N_PART = 100000
N_EDGE = 6400000
SUPPORT = 0.05
KGRAD = -20.0 * 7.0 / (math.pi * SUPPORT * SUPPORT) / SUPPORT

NC, NS, L = 2, 16, 16
NW = NC * NS
NP = 100352
RPT = NP // NS
LASTR = N_PART - 15 * RPT
EPW = N_EDGE // NW
CHUNK = 4000
NCHUNK = EPW // CHUNK
CSTEPS = CHUNK // L

_mesh = plsc.VectorSubcoreMesh(
    core_axis_name="c", subcore_axis_name="s", num_cores=NC, num_subcores=NS
)


@functools.partial(
    pl.kernel,
    out_type=jax.ShapeDtypeStruct((NC, NP), jnp.float32),
    mesh=_mesh,
    scratch_types=[
        pltpu.VMEM_SHARED((NP,), jnp.float32),
        pltpu.VMEM_SHARED((NP,), jnp.float32),
        pltpu.VMEM_SHARED((NP,), jnp.float32),
        pltpu.VMEM_SHARED((NP,), jnp.float32),
        pltpu.VMEM((2 * RPT,), jnp.float32),
        pltpu.VMEM((RPT,), jnp.float32),
        pltpu.VMEM((RPT,), jnp.float32),
        pltpu.VMEM((RPT,), jnp.float32),
        pltpu.VMEM((RPT,), jnp.float32),
        pltpu.VMEM((CHUNK,), jnp.int32),
        pltpu.VMEM((CHUNK,), jnp.int32),
        pltpu.VMEM((CHUNK,), jnp.float32),
        pltpu.VMEM((2 * CHUNK,), jnp.float32),
        pltpu.VMEM((CHUNK,), jnp.float32),
        pltpu.VMEM((CHUNK,), jnp.float32),
        pltpu.VMEM((CHUNK,), jnp.float32),
        pltpu.VMEM((CHUNK,), jnp.float32),
        pltpu.VMEM((CHUNK,), jnp.float32),
        pltpu.VMEM((CHUNK,), jnp.float32),
        pltpu.SemaphoreType.DMA,
    ],
    compiler_params=pltpu.CompilerParams(
        needs_layout_passes=False, use_tc_tiling_on_sc=False
    ),
)
def _sc_dpdt(ei_hbm, ej_hbm, v_hbm, rho_hbm, uflat_hbm, dist_hbm, rad_hbm,
             out_hbm,
             uxt, uyt, vrt, dpdt, uvf, vv, rv, uxb, uyb,
             ei, ej, rad, dst, uxj, uyj, vrj, uxi, uyi, cbuf, sem):
    cid = lax.axis_index("c")
    sid = lax.axis_index("s")
    wid = sid * NC + cid
    iota = lax.iota(jnp.int32, L)

    nbase = sid * RPT

    def stage(nrows):
        pltpu.sync_copy(uflat_hbm.at[pl.ds(2 * nbase, 2 * nrows)],
                        uvf.at[pl.ds(0, 2 * nrows)])
        pltpu.sync_copy(v_hbm.at[pl.ds(nbase, nrows)], vv.at[pl.ds(0, nrows)])
        pltpu.sync_copy(rho_hbm.at[pl.ds(nbase, nrows)], rv.at[pl.ds(0, nrows)])

        def build_step(k, carry):
            sl = pl.ds(k * L, L)
            rows2 = (k * L + iota) * 2
            uxb[sl] = plsc.load_gather(uvf, [rows2])
            uyb[sl] = plsc.load_gather(uvf, [rows2 + 1])
            rv[sl] = vv[sl] * rv[sl]
            return carry

        lax.fori_loop(0, nrows // L, build_step, 0)
        pltpu.sync_copy(uxb.at[pl.ds(0, nrows)], uxt.at[pl.ds(nbase, nrows)])
        pltpu.sync_copy(uyb.at[pl.ds(0, nrows)], uyt.at[pl.ds(nbase, nrows)])
        pltpu.sync_copy(rv.at[pl.ds(0, nrows)], vrt.at[pl.ds(nbase, nrows)])

    @pl.when(sid < NS - 1)
    def _():
        stage(RPT)

    @pl.when(sid == NS - 1)
    def _():
        stage(LASTR)

    def zero_step(k, carry):
        vv[pl.ds(k * L, L)] = jnp.zeros((L,), jnp.float32)
        return carry

    lax.fori_loop(0, RPT // L, zero_step, 0)
    pltpu.sync_copy(vv, dpdt.at[pl.ds(nbase, RPT)])

    plsc.subcore_barrier()

    ebase0 = wid * EPW

    def chunk_body(c, carry):
        eb = ebase0 + c * CHUNK
        pltpu.sync_copy(ei_hbm.at[pl.ds(eb, CHUNK)], ei)
        pltpu.sync_copy(ej_hbm.at[pl.ds(eb, CHUNK)], ej)
        pltpu.sync_copy(rad_hbm.at[pl.ds(eb, CHUNK)], rad)
        pltpu.sync_copy(dist_hbm.at[pl.ds(2 * eb, 2 * CHUNK)], dst)
        g1 = pltpu.async_copy(uxt.at[ej], uxj, sem)
        g2 = pltpu.async_copy(uyt.at[ej], uyj, sem)
        g3 = pltpu.async_copy(vrt.at[ej], vrj, sem)
        g4 = pltpu.async_copy(uxt.at[ei], uxi, sem)
        g5 = pltpu.async_copy(uyt.at[ei], uyi, sem)
        g1.wait()
        g2.wait()
        g3.wait()
        g4.wait()
        g5.wait()

        def step(k, inner):
            sl = pl.ds(k * L, L)
            rows = k * L + iota
            r = rad[sl]
            q = jnp.minimum(jnp.maximum(r, 0.0), 1.0)
            om = 1.0 - q
            w = (om * om) * (om * q) * KGRAD
            dx = plsc.load_gather(dst, [rows * 2])
            dy = plsc.load_gather(dst, [rows * 2 + 1])
            prod = (uxj[sl] - uxi[sl]) * dx + (uyj[sl] - uyi[sl]) * dy
            cbuf[sl] = prod * w * vrj[sl]
            return inner

        lax.fori_loop(0, CSTEPS, step, 0)
        pltpu.sync_copy(cbuf, dpdt.at[ei], add=True)
        return carry

    lax.fori_loop(0, NCHUNK, chunk_body, 0)

    plsc.subcore_barrier()
    pltpu.sync_copy(dpdt.at[pl.ds(nbase, RPT)],
                    out_hbm.at[cid, pl.ds(nbase, RPT)])


def _combine_body(p_ref, o_ref):
    o_ref[...] = -(p_ref[0, :N_PART] + p_ref[1, :N_PART])


_combine = pl.pallas_call(
    _combine_body,
    out_shape=jax.ShapeDtypeStruct((N_PART,), jnp.float32),
)


def kernel(edge_i, edge_j, V, rho, u, distances, radialDistances):
    ei = edge_i.astype(jnp.int32)
    ej = edge_j.astype(jnp.int32)
    u_flat = u.reshape(-1)
    dist_flat = distances.reshape(-1)
    part = _sc_dpdt(ei, ej, V, rho, u_flat, dist_flat, radialDistances)
    return _combine(part)


# speedup vs baseline: 13.4869x; 13.4869x over previous
import functools
import math

import jax
import jax.numpy as jnp
from jax import lax
from jax.experimental import pallas as pl
from jax.experimental.pallas import tpu as pltpu
from jax.experimental.pallas import tpu_sc as plsc

N_PART = 100000
N_EDGE = 6400000
SUPPORT = 0.05
KGRAD = -20.0 * 7.0 / (math.pi * SUPPORT * SUPPORT) / SUPPORT

NC, NS, L = 2, 16, 16
NW = NC * NS
B = 128
NP = 100352
RPT = NP // NS
LASTR = N_PART - 15 * RPT
EPW = N_EDGE // NW
CHUNK = 4000
NCHUNK = EPW // CHUNK
CSTEPS = CHUNK // L
DSLAB = (CHUNK // B + 2) * 2 * B

_mesh = plsc.VectorSubcoreMesh(
    core_axis_name="c", subcore_axis_name="s", num_cores=NC, num_subcores=NS
)


@functools.partial(
    pl.kernel,
    out_type=jax.ShapeDtypeStruct((NC, NP), jnp.float32),
    mesh=_mesh,
    scratch_types=[
        pltpu.VMEM_SHARED((NP,), jnp.float32),
        pltpu.VMEM_SHARED((NP,), jnp.float32),
        pltpu.VMEM_SHARED((NP,), jnp.float32),
        pltpu.VMEM_SHARED((NP,), jnp.float32),
        pltpu.VMEM((2 * RPT,), jnp.float32),
        pltpu.VMEM((RPT,), jnp.float32),
        pltpu.VMEM((RPT,), jnp.float32),
        pltpu.VMEM((RPT,), jnp.float32),
        pltpu.VMEM((RPT,), jnp.float32),
        pltpu.VMEM((CHUNK,), jnp.int32),
        pltpu.VMEM((CHUNK,), jnp.int32),
        pltpu.VMEM((CHUNK,), jnp.float32),
        pltpu.VMEM((DSLAB,), jnp.float32),
        pltpu.VMEM((CHUNK,), jnp.float32),
        pltpu.VMEM((CHUNK,), jnp.float32),
        pltpu.VMEM((CHUNK,), jnp.float32),
        pltpu.VMEM((CHUNK,), jnp.float32),
        pltpu.VMEM((CHUNK,), jnp.float32),
        pltpu.VMEM((CHUNK,), jnp.float32),
        pltpu.SemaphoreType.DMA,
    ],
    compiler_params=pltpu.CompilerParams(
        needs_layout_passes=False, use_tc_tiling_on_sc=False
    ),
)
def _sc_dpdt(ei_hbm, ej_hbm, v_hbm, rho_hbm, ublk_hbm, dblk_hbm, rad_hbm,
             out_hbm,
             uxt, uyt, vrt, dpdt, uvf, vv, rv, uxb, uyb,
             ei, ej, rad, dst, uxj, uyj, vrj, uxi, uyi, cbuf, sem):
    cid = lax.axis_index("c")
    sid = lax.axis_index("s")
    wid = sid * NC + cid
    iota = lax.iota(jnp.int32, L)

    nbase = sid * RPT

    pltpu.sync_copy(ublk_hbm.at[pl.ds(2 * nbase, 2 * RPT)], uvf)

    def ubuild_step(k, carry):
        sl = pl.ds(k * L, L)
        off = k * L + iota
        idx = ((off >> 7) << 8) + (off & (B - 1))
        uxb[sl] = plsc.load_gather(uvf, [idx])
        uyb[sl] = plsc.load_gather(uvf, [idx + B])
        return carry

    lax.fori_loop(0, RPT // L, ubuild_step, 0)
    pltpu.sync_copy(uxb, uxt.at[pl.ds(nbase, RPT)])
    pltpu.sync_copy(uyb, uyt.at[pl.ds(nbase, RPT)])

    def vrstage(nrows):
        pltpu.sync_copy(v_hbm.at[pl.ds(nbase, nrows)], vv.at[pl.ds(0, nrows)])
        pltpu.sync_copy(rho_hbm.at[pl.ds(nbase, nrows)], rv.at[pl.ds(0, nrows)])

        def vr_step(k, carry):
            sl = pl.ds(k * L, L)
            rv[sl] = vv[sl] * rv[sl]
            return carry

        lax.fori_loop(0, nrows // L, vr_step, 0)
        pltpu.sync_copy(rv.at[pl.ds(0, nrows)], vrt.at[pl.ds(nbase, nrows)])

    @pl.when(sid < NS - 1)
    def _():
        vrstage(RPT)

    @pl.when(sid == NS - 1)
    def _():
        vrstage(LASTR)

    def zero_step(k, carry):
        vv[pl.ds(k * L, L)] = jnp.zeros((L,), jnp.float32)
        return carry

    lax.fori_loop(0, RPT // L, zero_step, 0)
    pltpu.sync_copy(vv, dpdt.at[pl.ds(nbase, RPT)])

    plsc.subcore_barrier()

    ebase0 = wid * EPW

    def chunk_body(c, carry):
        eb = ebase0 + c * CHUNK
        pltpu.sync_copy(ei_hbm.at[pl.ds(eb, CHUNK)], ei)
        pltpu.sync_copy(ej_hbm.at[pl.ds(eb, CHUNK)], ej)
        pltpu.sync_copy(rad_hbm.at[pl.ds(eb, CHUNK)], rad)
        b0 = jnp.minimum(eb >> 7, N_EDGE // B - DSLAB // (2 * B))
        pltpu.sync_copy(dblk_hbm.at[pl.ds(b0 * (2 * B), DSLAB)], dst)
        off0 = eb - b0 * B
        g1 = pltpu.async_copy(uxt.at[ej], uxj, sem)
        g2 = pltpu.async_copy(uyt.at[ej], uyj, sem)
        g3 = pltpu.async_copy(vrt.at[ej], vrj, sem)
        g4 = pltpu.async_copy(uxt.at[ei], uxi, sem)
        g5 = pltpu.async_copy(uyt.at[ei], uyi, sem)
        g1.wait()
        g2.wait()
        g3.wait()
        g4.wait()
        g5.wait()

        def step(k, inner):
            sl = pl.ds(k * L, L)
            r = rad[sl]
            q = jnp.minimum(jnp.maximum(r, 0.0), 1.0)
            om = 1.0 - q
            w = (om * om) * (om * q) * KGRAD
            off = off0 + k * L + iota
            idx = ((off >> 7) << 8) + (off & (B - 1))
            dx = plsc.load_gather(dst, [idx])
            dy = plsc.load_gather(dst, [idx + B])
            prod = (uxj[sl] - uxi[sl]) * dx + (uyj[sl] - uyi[sl]) * dy
            cbuf[sl] = prod * w * vrj[sl]
            return inner

        lax.fori_loop(0, CSTEPS, step, 0)
        pltpu.sync_copy(cbuf, dpdt.at[ei], add=True)
        return carry

    lax.fori_loop(0, NCHUNK, chunk_body, 0)

    plsc.subcore_barrier()
    pltpu.sync_copy(dpdt.at[pl.ds(nbase, RPT)],
                    out_hbm.at[cid, pl.ds(nbase, RPT)])


def _combine_body(p_ref, o_ref):
    o_ref[...] = -(p_ref[0, :N_PART] + p_ref[1, :N_PART])


_combine = pl.pallas_call(
    _combine_body,
    out_shape=jax.ShapeDtypeStruct((N_PART,), jnp.float32),
)


def _planar_flat(x, nblocks):
    return x.reshape(nblocks, B, 2).transpose(0, 2, 1).reshape(-1)


def kernel(edge_i, edge_j, V, rho, u, distances, radialDistances):
    ei = edge_i.astype(jnp.int32)
    ej = edge_j.astype(jnp.int32)
    u_p = jnp.pad(u, ((0, NP - N_PART), (0, 0)))
    ublk = _planar_flat(u_p, NP // B)
    dblk = _planar_flat(distances, N_EDGE // B)
    part = _sc_dpdt(ei, ej, V, rho, ublk, dblk, radialDistances)
    return _combine(part)
